# Initial kernel scaffold; baseline (speedup 1.0000x reference)
import jax
import jax.numpy as jnp
from jax.experimental import pallas as pl


def kernel(x, edge_index, W1_l, b1_l, W1_r, W2_l, b2_l, W2_r):
    raise NotImplementedError("write your pallas kernel here")


# speedup vs baseline: 14.1099x; 14.1099x over previous
import functools

import jax
import jax.numpy as jnp
from jax import lax
from jax.experimental import pallas as pl
from jax.experimental.pallas import tpu as pltpu
from jax.experimental.pallas import tpu_sc as plsc

_NC = 2
_NS = 16
_K = 125


def _sc_aggregate(table, src2d, dst2d, zblk):
    n, w = table.shape
    rows_total, k = src2d.shape
    ch = rows_total // (_NC * _NS)
    per = (n // _NS) // 8 * 8
    last = n - (_NS - 1) * per
    mesh = plsc.VectorSubcoreMesh(core_axis_name="c", subcore_axis_name="s")

    @functools.partial(
        pl.kernel,
        out_type=jax.ShapeDtypeStruct((_NC, n, w), jnp.float32),
        mesh=mesh,
        scratch_types=[
            pltpu.VMEM((ch, k), jnp.int32),
            pltpu.VMEM((ch, k), jnp.int32),
            pltpu.VMEM((k, w), jnp.float32),
            pltpu.VMEM_SHARED((n, w), jnp.float32),
        ],
        compiler_params=pltpu.CompilerParams(use_tc_tiling_on_sc=False),
    )
    def agg(table_hbm, src_hbm, dst_hbm, z_hbm, out_hbm, sidx, didx, rows, acc):
        cid = lax.axis_index("c")
        sid = lax.axis_index("s")
        wid = cid * _NS + sid
        base = sid * per

        @pl.when(sid < _NS - 1)
        def _():
            pltpu.sync_copy(z_hbm.at[pl.ds(0, per)], acc.at[pl.ds(base, per)])

        @pl.when(sid == _NS - 1)
        def _():
            pltpu.sync_copy(z_hbm, acc.at[pl.ds(base, last)])

        pltpu.sync_copy(src_hbm.at[pl.ds(wid * ch, ch)], sidx)
        pltpu.sync_copy(dst_hbm.at[pl.ds(wid * ch, ch)], didx)
        plsc.subcore_barrier()

        @pl.loop(0, ch)
        def _(j):
            pltpu.sync_copy(table_hbm.at[sidx.at[j]], rows)
            pltpu.sync_copy(rows, acc.at[didx.at[j]], add=True)

        plsc.subcore_barrier()

        @pl.when(sid < _NS - 1)
        def _():
            pltpu.sync_copy(acc.at[pl.ds(base, per)],
                            out_hbm.at[cid, pl.ds(base, per)])

        @pl.when(sid == _NS - 1)
        def _():
            pltpu.sync_copy(acc.at[pl.ds(base, last)],
                            out_hbm.at[cid, pl.ds(base, last)])

    return agg(table, src2d, dst2d, zblk)


def _tc1_body(x_ref, wc_ref, table_ref, r1_ref):
    h = r1_ref.shape[-1]
    mm = jnp.dot(x_ref[...], wc_ref[...], preferred_element_type=jnp.float32)
    table_ref[:, :h] = mm[:, :h]
    table_ref[:, h:] = jnp.ones_like(mm[:, h:])
    r1_ref[...] = mm[:, h:]


def _tc2_body(acc_ref, r1_ref, b1_ref, h_ref):
    h = h_ref.shape[-1]
    a = acc_ref[...]
    s = a[0] + a[1]
    cnt = jnp.maximum(s[:, h:h + 1], 1.0)
    h_ref[...] = jnp.maximum(s[:, :h] / cnt + b1_ref[...] + r1_ref[...], 0.0)


def _tc3_body(acc2_ref, acc1_ref, h_ref, w2l_ref, w2r_ref, b2_ref, o_ref):
    hd = h_ref.shape[-1]
    a2 = acc2_ref[...]
    s2 = a2[0] + a2[1]
    a1 = acc_lane = acc1_ref[...]
    cnt = jnp.maximum(a1[0, :, hd:hd + 1] + a1[1, :, hd:hd + 1], 1.0)
    z = (jnp.dot(s2 / cnt, w2l_ref[...], preferred_element_type=jnp.float32)
         + jnp.dot(h_ref[...], w2r_ref[...], preferred_element_type=jnp.float32)
         + b2_ref[...])
    z = z - jnp.max(z, axis=1, keepdims=True)
    o_ref[...] = z - jnp.log(jnp.sum(jnp.exp(z), axis=1, keepdims=True))


def kernel(x, edge_index, W1_l, b1_l, W1_r, W2_l, b2_l, W2_r):
    n, d = x.shape
    e = edge_index.shape[1]
    h = W1_l.shape[0]
    c = W2_l.shape[0]
    bn = 2000

    src2d = edge_index[0].reshape(e // _K, _K)
    dst2d = edge_index[1].reshape(e // _K, _K)
    zrows = n - (_NS - 1) * ((n // _NS) // 8 * 8)
    z1 = jnp.zeros((zrows, 2 * h), jnp.float32)
    z2 = jnp.zeros((zrows, h), jnp.float32)
    wc = jnp.concatenate([W1_l.T, W1_r.T], axis=1)
    w2l_t = W2_l.T
    w2r_t = W2_r.T
    b1 = b1_l.reshape(1, h)
    b2 = b2_l.reshape(1, c)

    grid = (n // bn,)
    full = lambda shape: pl.BlockSpec(shape, lambda i: (0,) * len(shape))
    rowblk = lambda mnr: pl.BlockSpec((bn, mnr), lambda i: (i, 0))
    accblk = lambda mnr: pl.BlockSpec((_NC, bn, mnr), lambda i: (0, i, 0))

    table1, r1 = pl.pallas_call(
        _tc1_body,
        grid=grid,
        in_specs=[rowblk(d), full((d, 2 * h))],
        out_specs=[rowblk(2 * h), rowblk(h)],
        out_shape=[
            jax.ShapeDtypeStruct((n, 2 * h), jnp.float32),
            jax.ShapeDtypeStruct((n, h), jnp.float32),
        ],
    )(x, wc)

    acc1 = _sc_aggregate(table1, src2d, dst2d, z1)

    hfeat = pl.pallas_call(
        _tc2_body,
        grid=grid,
        in_specs=[accblk(2 * h), rowblk(h), full((1, h))],
        out_specs=rowblk(h),
        out_shape=jax.ShapeDtypeStruct((n, h), jnp.float32),
    )(acc1, r1, b1)

    acc2 = _sc_aggregate(hfeat, src2d, dst2d, z2)

    out = pl.pallas_call(
        _tc3_body,
        grid=grid,
        in_specs=[accblk(h), accblk(2 * h), rowblk(h), full((h, c)),
                  full((h, c)), full((1, c))],
        out_specs=rowblk(c),
        out_shape=jax.ShapeDtypeStruct((n, c), jnp.float32),
    )(acc2, acc1, hfeat, w2l_t, w2r_t, b2)

    return out


# speedup vs baseline: 15.6614x; 1.1100x over previous
import functools

import jax
import jax.numpy as jnp
from jax import lax
from jax.experimental import pallas as pl
from jax.experimental.pallas import tpu as pltpu
from jax.experimental.pallas import tpu_sc as plsc

_NC = 2
_NS = 16
_K = 125


def _sc_aggregate(table, src2d, dst2d, zblk):
    n, w = table.shape
    rows_total, k = src2d.shape
    ch = rows_total // (_NC * _NS)
    per = (n // _NS) // 8 * 8
    last = n - (_NS - 1) * per
    mesh = plsc.VectorSubcoreMesh(core_axis_name="c", subcore_axis_name="s")

    @functools.partial(
        pl.kernel,
        out_type=jax.ShapeDtypeStruct((_NC, n, w), jnp.float32),
        mesh=mesh,
        scratch_types=[
            pltpu.VMEM((ch, k), jnp.int32),
            pltpu.VMEM((ch, k), jnp.int32),
            pltpu.VMEM((k, w), jnp.float32),
            pltpu.VMEM((k, w), jnp.float32),
            pltpu.VMEM_SHARED((n, w), jnp.float32),
            pltpu.SemaphoreType.DMA,
            pltpu.SemaphoreType.DMA,
        ],
        compiler_params=pltpu.CompilerParams(use_tc_tiling_on_sc=False),
    )
    def agg(table_hbm, src_hbm, dst_hbm, z_hbm, out_hbm, sidx, didx,
            rows0, rows1, acc, gs0, gs1):
        cid = lax.axis_index("c")
        sid = lax.axis_index("s")
        wid = cid * _NS + sid
        base = sid * per

        @pl.when(sid < _NS - 1)
        def _():
            pltpu.sync_copy(z_hbm.at[pl.ds(0, per)], acc.at[pl.ds(base, per)])

        @pl.when(sid == _NS - 1)
        def _():
            pltpu.sync_copy(z_hbm, acc.at[pl.ds(base, last)])

        pltpu.sync_copy(src_hbm.at[pl.ds(wid * ch, ch)], sidx)
        pltpu.sync_copy(dst_hbm.at[pl.ds(wid * ch, ch)], didx)
        plsc.subcore_barrier()

        pltpu.async_copy(table_hbm.at[sidx.at[0]], rows0, gs0).wait()

        @pl.loop(0, ch, step=2)
        def _(j):
            pltpu.async_copy(table_hbm.at[sidx.at[j + 1]], rows1, gs1)
            pltpu.sync_copy(rows0, acc.at[didx.at[j]], add=True)
            pltpu.make_async_copy(table_hbm.at[sidx.at[j + 1]], rows1, gs1).wait()

            @pl.when(j + 2 < ch)
            def _():
                pltpu.async_copy(table_hbm.at[sidx.at[j + 2]], rows0, gs0)

            pltpu.sync_copy(rows1, acc.at[didx.at[j + 1]], add=True)

            @pl.when(j + 2 < ch)
            def _():
                pltpu.make_async_copy(table_hbm.at[sidx.at[j + 2]], rows0, gs0).wait()

        plsc.subcore_barrier()

        @pl.when(sid < _NS - 1)
        def _():
            pltpu.sync_copy(acc.at[pl.ds(base, per)],
                            out_hbm.at[cid, pl.ds(base, per)])

        @pl.when(sid == _NS - 1)
        def _():
            pltpu.sync_copy(acc.at[pl.ds(base, last)],
                            out_hbm.at[cid, pl.ds(base, last)])

    return agg(table, src2d, dst2d, zblk)


def _tc1_body(x_ref, wc_ref, table_ref, r1_ref):
    h = r1_ref.shape[-1]
    mm = jnp.dot(x_ref[...], wc_ref[...], preferred_element_type=jnp.float32)
    table_ref[:, :h] = mm[:, :h]
    table_ref[:, h:] = jnp.ones_like(mm[:, h:])
    r1_ref[...] = mm[:, h:]


def _tc2_body(acc_ref, r1_ref, b1_ref, h_ref):
    h = h_ref.shape[-1]
    a = acc_ref[...]
    s = a[0] + a[1]
    cnt = jnp.maximum(s[:, h:h + 1], 1.0)
    h_ref[...] = jnp.maximum(s[:, :h] / cnt + b1_ref[...] + r1_ref[...], 0.0)


def _tc3_body(acc2_ref, acc1_ref, h_ref, w2l_ref, w2r_ref, b2_ref, o_ref):
    hd = h_ref.shape[-1]
    a2 = acc2_ref[...]
    s2 = a2[0] + a2[1]
    a1 = acc_lane = acc1_ref[...]
    cnt = jnp.maximum(a1[0, :, hd:hd + 1] + a1[1, :, hd:hd + 1], 1.0)
    z = (jnp.dot(s2 / cnt, w2l_ref[...], preferred_element_type=jnp.float32)
         + jnp.dot(h_ref[...], w2r_ref[...], preferred_element_type=jnp.float32)
         + b2_ref[...])
    z = z - jnp.max(z, axis=1, keepdims=True)
    o_ref[...] = z - jnp.log(jnp.sum(jnp.exp(z), axis=1, keepdims=True))


def kernel(x, edge_index, W1_l, b1_l, W1_r, W2_l, b2_l, W2_r):
    n, d = x.shape
    e = edge_index.shape[1]
    h = W1_l.shape[0]
    c = W2_l.shape[0]
    bn = 2000

    src2d = edge_index[0].reshape(e // _K, _K)
    dst2d = edge_index[1].reshape(e // _K, _K)
    zrows = n - (_NS - 1) * ((n // _NS) // 8 * 8)
    z1 = jnp.zeros((zrows, 2 * h), jnp.float32)
    z2 = jnp.zeros((zrows, h), jnp.float32)
    wc = jnp.concatenate([W1_l.T, W1_r.T], axis=1)
    w2l_t = W2_l.T
    w2r_t = W2_r.T
    b1 = b1_l.reshape(1, h)
    b2 = b2_l.reshape(1, c)

    grid = (n // bn,)
    full = lambda shape: pl.BlockSpec(shape, lambda i: (0,) * len(shape))
    rowblk = lambda mnr: pl.BlockSpec((bn, mnr), lambda i: (i, 0))
    accblk = lambda mnr: pl.BlockSpec((_NC, bn, mnr), lambda i: (0, i, 0))

    table1, r1 = pl.pallas_call(
        _tc1_body,
        grid=grid,
        in_specs=[rowblk(d), full((d, 2 * h))],
        out_specs=[rowblk(2 * h), rowblk(h)],
        out_shape=[
            jax.ShapeDtypeStruct((n, 2 * h), jnp.float32),
            jax.ShapeDtypeStruct((n, h), jnp.float32),
        ],
    )(x, wc)

    acc1 = _sc_aggregate(table1, src2d, dst2d, z1)

    hfeat = pl.pallas_call(
        _tc2_body,
        grid=grid,
        in_specs=[accblk(2 * h), rowblk(h), full((1, h))],
        out_specs=rowblk(h),
        out_shape=jax.ShapeDtypeStruct((n, h), jnp.float32),
    )(acc1, r1, b1)

    acc2 = _sc_aggregate(hfeat, src2d, dst2d, z2)

    out = pl.pallas_call(
        _tc3_body,
        grid=grid,
        in_specs=[accblk(h), accblk(2 * h), rowblk(h), full((h, c)),
                  full((h, c)), full((1, c))],
        out_specs=rowblk(c),
        out_shape=jax.ShapeDtypeStruct((n, c), jnp.float32),
    )(acc2, acc1, hfeat, w2l_t, w2r_t, b2)

    return out


# speedup vs baseline: 22.4497x; 1.4334x over previous
import functools

import jax
import jax.numpy as jnp
from jax import lax
from jax.experimental import pallas as pl
from jax.experimental.pallas import tpu as pltpu
from jax.experimental.pallas import tpu_sc as plsc

_NC = 2
_NS = 16
_K = 125
_Q = 4


def _sc_aggregate(table, src2d, dst2d, zblk):
    n, w = table.shape
    rows_total, k = src2d.shape
    ch = rows_total // (_NC * _NS)
    per = (n // _NS) // 8 * 8
    last = n - (_NS - 1) * per
    mesh = plsc.VectorSubcoreMesh(core_axis_name="c", subcore_axis_name="s")

    @functools.partial(
        pl.kernel,
        out_type=jax.ShapeDtypeStruct((_NC, n, w), jnp.float32),
        mesh=mesh,
        scratch_types=[
            pltpu.VMEM((ch, k), jnp.int32),
            pltpu.VMEM((ch, k), jnp.int32),
            pltpu.VMEM((_Q, k, w), jnp.float32),
            pltpu.SemaphoreType.DMA((_Q,)),
            pltpu.SemaphoreType.DMA((_Q,)),
        ] + [pltpu.VMEM_SHARED((n, w), jnp.float32)],
        compiler_params=pltpu.CompilerParams(use_tc_tiling_on_sc=False),
    )
    def agg(table_hbm, src_hbm, dst_hbm, z_hbm, out_hbm, sidx, didx,
            rows, gsem, ssem, acc):
        cid = lax.axis_index("c")
        sid = lax.axis_index("s")
        wid = cid * _NS + sid
        base = sid * per

        @pl.when(sid < _NS - 1)
        def _():
            pltpu.sync_copy(z_hbm.at[pl.ds(0, per)], acc.at[pl.ds(base, per)])

        @pl.when(sid == _NS - 1)
        def _():
            pltpu.sync_copy(z_hbm, acc.at[pl.ds(base, last)])

        pltpu.sync_copy(src_hbm.at[pl.ds(wid * ch, ch)], sidx)
        pltpu.sync_copy(dst_hbm.at[pl.ds(wid * ch, ch)], didx)
        plsc.subcore_barrier()

        for b in range(_Q):
            pltpu.async_copy(table_hbm.at[sidx.at[b]], rows.at[b], gsem.at[b])

        @pl.loop(0, ch, step=_Q)
        def _(j):
            descs = []
            for b in range(_Q):
                pltpu.make_async_copy(table_hbm.at[sidx.at[j + b]],
                                      rows.at[b], gsem.at[b]).wait()
                descs.append(pltpu.async_copy(
                    rows.at[b], acc.at[didx.at[j + b]], ssem.at[b], add=True))
            for b in range(_Q):
                @pl.when(j + _Q + b < ch)
                def _(b=b):
                    descs[b].wait()
                    pltpu.async_copy(table_hbm.at[sidx.at[j + _Q + b]],
                                     rows.at[b], gsem.at[b])

        for b in range(_Q):
            pltpu.make_async_copy(rows.at[b], acc.at[didx.at[ch - _Q + b]],
                                  ssem.at[b]).wait()
        plsc.subcore_barrier()

        @pl.when(sid < _NS - 1)
        def _():
            pltpu.sync_copy(acc.at[pl.ds(base, per)],
                            out_hbm.at[cid, pl.ds(base, per)])

        @pl.when(sid == _NS - 1)
        def _():
            pltpu.sync_copy(acc.at[pl.ds(base, last)],
                            out_hbm.at[cid, pl.ds(base, last)])

    return agg(table, src2d, dst2d, zblk)


def _tc1_body(x_ref, wc_ref, table_ref, r1_ref):
    h = r1_ref.shape[-1]
    mm = jnp.dot(x_ref[...], wc_ref[...], preferred_element_type=jnp.float32)
    table_ref[:, :h] = mm[:, :h]
    table_ref[:, h:] = jnp.ones_like(mm[:, h:])
    r1_ref[...] = mm[:, h:]


def _tc2_body(acc_ref, r1_ref, b1_ref, h_ref):
    h = h_ref.shape[-1]
    a = acc_ref[...]
    s = a[0] + a[1]
    cnt = jnp.maximum(s[:, h:h + 1], 1.0)
    h_ref[...] = jnp.maximum(s[:, :h] / cnt + b1_ref[...] + r1_ref[...], 0.0)


def _tc3_body(acc2_ref, acc1_ref, h_ref, w2l_ref, w2r_ref, b2_ref, o_ref):
    hd = h_ref.shape[-1]
    a2 = acc2_ref[...]
    s2 = a2[0] + a2[1]
    a1 = acc_lane = acc1_ref[...]
    cnt = jnp.maximum(a1[0, :, hd:hd + 1] + a1[1, :, hd:hd + 1], 1.0)
    z = (jnp.dot(s2 / cnt, w2l_ref[...], preferred_element_type=jnp.float32)
         + jnp.dot(h_ref[...], w2r_ref[...], preferred_element_type=jnp.float32)
         + b2_ref[...])
    z = z - jnp.max(z, axis=1, keepdims=True)
    o_ref[...] = z - jnp.log(jnp.sum(jnp.exp(z), axis=1, keepdims=True))


def kernel(x, edge_index, W1_l, b1_l, W1_r, W2_l, b2_l, W2_r):
    n, d = x.shape
    e = edge_index.shape[1]
    h = W1_l.shape[0]
    c = W2_l.shape[0]
    bn = 2000

    src2d = edge_index[0].reshape(e // _K, _K)
    dst2d = edge_index[1].reshape(e // _K, _K)
    zrows = n - (_NS - 1) * ((n // _NS) // 8 * 8)
    z1 = jnp.zeros((zrows, 2 * h), jnp.float32)
    z2 = jnp.zeros((zrows, h), jnp.float32)
    wc = jnp.concatenate([W1_l.T, W1_r.T], axis=1)
    w2l_t = W2_l.T
    w2r_t = W2_r.T
    b1 = b1_l.reshape(1, h)
    b2 = b2_l.reshape(1, c)

    grid = (n // bn,)
    full = lambda shape: pl.BlockSpec(shape, lambda i: (0,) * len(shape))
    rowblk = lambda mnr: pl.BlockSpec((bn, mnr), lambda i: (i, 0))
    accblk = lambda mnr: pl.BlockSpec((_NC, bn, mnr), lambda i: (0, i, 0))

    table1, r1 = pl.pallas_call(
        _tc1_body,
        grid=grid,
        in_specs=[rowblk(d), full((d, 2 * h))],
        out_specs=[rowblk(2 * h), rowblk(h)],
        out_shape=[
            jax.ShapeDtypeStruct((n, 2 * h), jnp.float32),
            jax.ShapeDtypeStruct((n, h), jnp.float32),
        ],
    )(x, wc)

    acc1 = _sc_aggregate(table1, src2d, dst2d, z1)

    hfeat = pl.pallas_call(
        _tc2_body,
        grid=grid,
        in_specs=[accblk(2 * h), rowblk(h), full((1, h))],
        out_specs=rowblk(h),
        out_shape=jax.ShapeDtypeStruct((n, h), jnp.float32),
    )(acc1, r1, b1)

    acc2 = _sc_aggregate(hfeat, src2d, dst2d, z2)

    out = pl.pallas_call(
        _tc3_body,
        grid=grid,
        in_specs=[accblk(h), accblk(2 * h), rowblk(h), full((h, c)),
                  full((h, c)), full((1, c))],
        out_specs=rowblk(c),
        out_shape=jax.ShapeDtypeStruct((n, c), jnp.float32),
    )(acc2, acc1, hfeat, w2l_t, w2r_t, b2)

    return out


# speedup vs baseline: 23.9384x; 1.0663x over previous
import functools

import jax
import jax.numpy as jnp
from jax import lax
from jax.experimental import pallas as pl
from jax.experimental.pallas import tpu as pltpu
from jax.experimental.pallas import tpu_sc as plsc

_NC = 2
_NS = 16
_K = 128
_Q = 6


def _sc_aggregate(table2d, src, dst, z2d, n, w):
    nch = src.shape[0]
    ch = nch // (_NC * _NS)
    nx = nch - ch * _NC * _NS
    chm = ch // _Q * _Q
    per = (n // _NS) // 8 * 8
    last = n - (_NS - 1) * per
    mesh = plsc.VectorSubcoreMesh(core_axis_name="c", subcore_axis_name="s")

    @functools.partial(
        pl.kernel,
        out_type=jax.ShapeDtypeStruct((_NC, n, w), jnp.float32),
        mesh=mesh,
        scratch_types=[
            pltpu.VMEM((ch + 1, _K), jnp.int32),
            pltpu.VMEM((ch + 1, _K), jnp.int32),
            pltpu.VMEM((_Q, _K, w), jnp.float32),
            pltpu.SemaphoreType.DMA((_Q,)),
            pltpu.SemaphoreType.DMA((_Q,)),
            pltpu.VMEM_SHARED((n, w), jnp.float32),
        ],
        compiler_params=pltpu.CompilerParams(use_tc_tiling_on_sc=False),
    )
    def agg(table_hbm, src_hbm, dst_hbm, z_hbm, out_hbm, sidx, didx,
            rows, gsem, ssem, acc):
        cid = lax.axis_index("c")
        sid = lax.axis_index("s")
        wid = cid * _NS + sid
        base = sid * per
        tbl = table_hbm
        out = out_hbm
        zr = z_hbm
        src2 = src_hbm
        dst2 = dst_hbm

        @pl.when(sid < _NS - 1)
        def _():
            pltpu.sync_copy(zr.at[pl.ds(0, per)], acc.at[pl.ds(base, per)])

        @pl.when(sid == _NS - 1)
        def _():
            pltpu.sync_copy(zr, acc.at[pl.ds(base, last)])

        pltpu.sync_copy(src2.at[pl.ds(wid * ch, ch)], sidx.at[pl.ds(0, ch)])
        pltpu.sync_copy(dst2.at[pl.ds(wid * ch, ch)], didx.at[pl.ds(0, ch)])

        @pl.when(wid < nx)
        def _():
            xrow = _NC * _NS * ch + wid
            pltpu.sync_copy(src2.at[pl.ds(xrow, 1)], sidx.at[pl.ds(ch, 1)])
            pltpu.sync_copy(dst2.at[pl.ds(xrow, 1)], didx.at[pl.ds(ch, 1)])

        plsc.subcore_barrier()

        for b in range(_Q):
            pltpu.async_copy(tbl.at[sidx.at[b]], rows.at[b], gsem.at[b])

        @pl.loop(0, chm, step=_Q)
        def _(j):
            descs = []
            for b in range(_Q):
                pltpu.make_async_copy(tbl.at[sidx.at[j + b]],
                                      rows.at[b], gsem.at[b]).wait()
                descs.append(pltpu.async_copy(
                    rows.at[b], acc.at[didx.at[j + b]], ssem.at[b], add=True))
            for b in range(_Q):
                @pl.when(j + _Q + b < chm)
                def _(b=b):
                    descs[b].wait()
                    pltpu.async_copy(tbl.at[sidx.at[j + _Q + b]],
                                     rows.at[b], gsem.at[b])

        for b in range(_Q):
            pltpu.make_async_copy(rows.at[b], acc.at[didx.at[chm - _Q + b]],
                                  ssem.at[b]).wait()

        @pl.loop(chm, ch)
        def _(j):
            pltpu.sync_copy(tbl.at[sidx.at[j]], rows.at[0])
            pltpu.sync_copy(rows.at[0], acc.at[didx.at[j]], add=True)

        @pl.when(wid < nx)
        def _():
            pltpu.sync_copy(tbl.at[sidx.at[ch]], rows.at[1])
            pltpu.sync_copy(rows.at[1], acc.at[didx.at[ch]], add=True)

        plsc.subcore_barrier()

        @pl.when(sid < _NS - 1)
        def _():
            pltpu.sync_copy(acc.at[pl.ds(base, per)],
                            out.at[cid, pl.ds(base, per)])

        @pl.when(sid == _NS - 1)
        def _():
            pltpu.sync_copy(acc.at[pl.ds(base, last)],
                            out.at[cid, pl.ds(base, last)])

    return agg(table2d, src, dst, z2d)


def _tc1_body(x_ref, wl_ref, wr_ref, table_ref, r1_ref):
    dims = (((1,), (1,)), ((), ()))
    t1 = lax.dot_general(x_ref[...], wl_ref[...], dims,
                         preferred_element_type=jnp.float32)
    r1 = lax.dot_general(x_ref[...], wr_ref[...], dims,
                         preferred_element_type=jnp.float32)
    table_ref[...] = jnp.concatenate([t1, jnp.ones_like(t1)], axis=1)
    r1_ref[...] = r1


def _tc2_body(acc_ref, r1_ref, b1_ref, h_ref):
    h = r1_ref.shape[-1]
    a = acc_ref[...]
    s = a[0] + a[1]
    cnt = jnp.maximum(s[:, h:h + 1], 1.0)
    h_ref[...] = jnp.maximum(s[:, :h] / cnt + b1_ref[...][None, :]
                             + r1_ref[...], 0.0)


def _tc3_body(acc2_ref, acc1_ref, h_ref, w2l_ref, w2r_ref, b2_ref, o_ref):
    c, hd = w2l_ref.shape
    a2 = acc2_ref[...]
    s2 = a2[0] + a2[1]
    a1 = acc1_ref[...]
    cnt = jnp.maximum(a1[0, :, hd:hd + 1] + a1[1, :, hd:hd + 1], 1.0)
    hv = h_ref[...]
    dims = (((1,), (1,)), ((), ()))
    z = (lax.dot_general(s2 / cnt, w2l_ref[...], dims,
                         preferred_element_type=jnp.float32)
         + lax.dot_general(hv, w2r_ref[...], dims,
                           preferred_element_type=jnp.float32)
         + b2_ref[...][None, :])
    z = z - jnp.max(z, axis=1, keepdims=True)
    o_ref[...] = z - jnp.log(jnp.sum(jnp.exp(z), axis=1, keepdims=True))


def kernel(x, edge_index, W1_l, b1_l, W1_r, W2_l, b2_l, W2_r):
    n, d = x.shape
    h = W1_l.shape[0]
    c = W2_l.shape[0]

    e = edge_index.shape[1]
    src = edge_index[0].reshape(e // _K, _K)
    dst = edge_index[1].reshape(e // _K, _K)
    zrows = n - (_NS - 1) * ((n // _NS) // 8 * 8)
    z1 = jnp.zeros((zrows, 2 * h), jnp.float32)
    z2 = jnp.zeros((zrows, h), jnp.float32)

    table1, r1 = pl.pallas_call(
        _tc1_body,
        out_shape=[
            jax.ShapeDtypeStruct((n, 2 * h), jnp.float32),
            jax.ShapeDtypeStruct((n, h), jnp.float32),
        ],
    )(x, W1_l, W1_r)

    acc1 = _sc_aggregate(table1, src, dst, z1, n, 2 * h)

    hfeat = pl.pallas_call(
        _tc2_body,
        out_shape=jax.ShapeDtypeStruct((n, h), jnp.float32),
    )(acc1, r1, b1_l)

    acc2 = _sc_aggregate(hfeat, src, dst, z2, n, h)

    out = pl.pallas_call(
        _tc3_body,
        out_shape=jax.ShapeDtypeStruct((n, c), jnp.float32),
    )(acc2, acc1, hfeat, W2_l, W2_r, b2_l)

    return out


# speedup vs baseline: 25.6925x; 1.0733x over previous
import functools

import jax
import jax.numpy as jnp
from jax import lax
from jax.experimental import pallas as pl
from jax.experimental.pallas import tpu as pltpu
from jax.experimental.pallas import tpu_sc as plsc

_NC = 2
_NS = 16
_K = 128
_Q = 6


def _sc_aggregate(table2d, ei3, z2d, n, w):
    nch = ei3.shape[1]
    ch = nch // (_NC * _NS)
    nx = nch - ch * _NC * _NS
    chm = ch // _Q * _Q
    per = (n // _NS) // 8 * 8
    last = n - (_NS - 1) * per
    mesh = plsc.VectorSubcoreMesh(core_axis_name="c", subcore_axis_name="s")

    @functools.partial(
        pl.kernel,
        out_type=jax.ShapeDtypeStruct((_NC, n, w), jnp.float32),
        mesh=mesh,
        scratch_types=[
            pltpu.VMEM((ch + 1, _K), jnp.int32),
            pltpu.VMEM((ch + 1, _K), jnp.int32),
            pltpu.VMEM((_Q, _K, w), jnp.float32),
            pltpu.SemaphoreType.DMA((_Q,)),
            pltpu.SemaphoreType.DMA((_Q,)),
            pltpu.VMEM_SHARED((n, w), jnp.float32),
        ],
        compiler_params=pltpu.CompilerParams(use_tc_tiling_on_sc=False),
    )
    def agg(table_hbm, ei_hbm, z_hbm, out_hbm, sidx, didx,
            rows, gsem, ssem, acc):
        cid = lax.axis_index("c")
        sid = lax.axis_index("s")
        wid = cid * _NS + sid
        base = sid * per
        tbl = table_hbm
        out = out_hbm
        zr = z_hbm
        src2 = ei_hbm.at[0]
        dst2 = ei_hbm.at[1]

        @pl.when(sid < _NS - 1)
        def _():
            pltpu.sync_copy(zr.at[pl.ds(0, per)], acc.at[pl.ds(base, per)])

        @pl.when(sid == _NS - 1)
        def _():
            pltpu.sync_copy(zr, acc.at[pl.ds(base, last)])

        pltpu.sync_copy(src2.at[pl.ds(wid * ch, ch)], sidx.at[pl.ds(0, ch)])
        pltpu.sync_copy(dst2.at[pl.ds(wid * ch, ch)], didx.at[pl.ds(0, ch)])

        @pl.when(wid < nx)
        def _():
            xrow = _NC * _NS * ch + wid
            pltpu.sync_copy(src2.at[pl.ds(xrow, 1)], sidx.at[pl.ds(ch, 1)])
            pltpu.sync_copy(dst2.at[pl.ds(xrow, 1)], didx.at[pl.ds(ch, 1)])

        plsc.subcore_barrier()

        for b in range(_Q):
            pltpu.async_copy(tbl.at[sidx.at[b]], rows.at[b], gsem.at[b])

        @pl.loop(0, chm, step=_Q)
        def _(j):
            descs = []
            for b in range(_Q):
                pltpu.make_async_copy(tbl.at[sidx.at[j + b]],
                                      rows.at[b], gsem.at[b]).wait()
                descs.append(pltpu.async_copy(
                    rows.at[b], acc.at[didx.at[j + b]], ssem.at[b], add=True))
            for b in range(_Q):
                @pl.when(j + _Q + b < chm)
                def _(b=b):
                    descs[b].wait()
                    pltpu.async_copy(tbl.at[sidx.at[j + _Q + b]],
                                     rows.at[b], gsem.at[b])

        for b in range(_Q):
            pltpu.make_async_copy(rows.at[b], acc.at[didx.at[chm - _Q + b]],
                                  ssem.at[b]).wait()

        @pl.loop(chm, ch)
        def _(j):
            pltpu.sync_copy(tbl.at[sidx.at[j]], rows.at[0])
            pltpu.sync_copy(rows.at[0], acc.at[didx.at[j]], add=True)

        @pl.when(wid < nx)
        def _():
            pltpu.sync_copy(tbl.at[sidx.at[ch]], rows.at[1])
            pltpu.sync_copy(rows.at[1], acc.at[didx.at[ch]], add=True)

        plsc.subcore_barrier()

        @pl.when(sid < _NS - 1)
        def _():
            pltpu.sync_copy(acc.at[pl.ds(base, per)],
                            out.at[cid, pl.ds(base, per)])

        @pl.when(sid == _NS - 1)
        def _():
            pltpu.sync_copy(acc.at[pl.ds(base, last)],
                            out.at[cid, pl.ds(base, last)])

    return agg(table2d, ei3, z2d)


def _tc1_body(x_ref, wl_ref, wr_ref, table_ref, r1_ref):
    dims = (((1,), (1,)), ((), ()))
    t1 = lax.dot_general(x_ref[...], wl_ref[...], dims,
                         preferred_element_type=jnp.float32)
    r1 = lax.dot_general(x_ref[...], wr_ref[...], dims,
                         preferred_element_type=jnp.float32)
    table_ref[...] = jnp.concatenate([t1, jnp.ones_like(t1)], axis=1)
    r1_ref[...] = r1


def _tc2_body(acc_ref, r1_ref, b1_ref, h_ref):
    h = r1_ref.shape[-1]
    a = acc_ref[...]
    s = a[0] + a[1]
    cnt = jnp.maximum(s[:, h:h + 1], 1.0)
    h_ref[...] = jnp.maximum(s[:, :h] / cnt + b1_ref[...][None, :]
                             + r1_ref[...], 0.0)


def _tc3_body(acc2_ref, acc1_ref, h_ref, w2l_ref, w2r_ref, b2_ref, o_ref):
    c, hd = w2l_ref.shape
    a2 = acc2_ref[...]
    s2 = a2[0] + a2[1]
    a1 = acc1_ref[...]
    cnt = jnp.maximum(a1[0, :, hd:hd + 1] + a1[1, :, hd:hd + 1], 1.0)
    hv = h_ref[...]
    dims = (((1,), (1,)), ((), ()))
    z = (lax.dot_general(s2 / cnt, w2l_ref[...], dims,
                         preferred_element_type=jnp.float32)
         + lax.dot_general(hv, w2r_ref[...], dims,
                           preferred_element_type=jnp.float32)
         + b2_ref[...][None, :])
    z = z - jnp.max(z, axis=1, keepdims=True)
    o_ref[...] = z - jnp.log(jnp.sum(jnp.exp(z), axis=1, keepdims=True))


def kernel(x, edge_index, W1_l, b1_l, W1_r, W2_l, b2_l, W2_r):
    n, d = x.shape
    h = W1_l.shape[0]
    c = W2_l.shape[0]

    e = edge_index.shape[1]
    ei3 = edge_index.reshape(2, e // _K, _K)
    zrows = n - (_NS - 1) * ((n // _NS) // 8 * 8)
    z1 = jnp.zeros((zrows, 2 * h), jnp.float32)
    z2 = jnp.zeros((zrows, h), jnp.float32)

    bn = 2000
    grid = (n // bn,)
    full = lambda shape: pl.BlockSpec(shape, lambda i: (0,) * len(shape))
    rowblk = lambda mnr: pl.BlockSpec((bn, mnr), lambda i: (i, 0))
    accblk = lambda mnr: pl.BlockSpec((_NC, bn, mnr), lambda i: (0, i, 0))

    table1, r1 = pl.pallas_call(
        _tc1_body,
        grid=grid,
        in_specs=[rowblk(d), full((h, d)), full((h, d))],
        out_specs=[rowblk(2 * h), rowblk(h)],
        out_shape=[
            jax.ShapeDtypeStruct((n, 2 * h), jnp.float32),
            jax.ShapeDtypeStruct((n, h), jnp.float32),
        ],
    )(x, W1_l, W1_r)

    acc1 = _sc_aggregate(table1, ei3, z1, n, 2 * h)

    hfeat = pl.pallas_call(
        _tc2_body,
        grid=grid,
        in_specs=[accblk(2 * h), rowblk(h), full((h,))],
        out_specs=rowblk(h),
        out_shape=jax.ShapeDtypeStruct((n, h), jnp.float32),
    )(acc1, r1, b1_l)

    acc2 = _sc_aggregate(hfeat, ei3, z2, n, h)

    out = pl.pallas_call(
        _tc3_body,
        grid=grid,
        in_specs=[accblk(h), accblk(2 * h), rowblk(h),
                  full((c, h)), full((c, h)), full((c,))],
        out_specs=rowblk(c),
        out_shape=jax.ShapeDtypeStruct((n, c), jnp.float32),
    )(acc2, acc1, hfeat, W2_l, W2_r, b2_l)

    return out


# speedup vs baseline: 26.4490x; 1.0294x over previous
import functools

import jax
import jax.numpy as jnp
from jax import lax
from jax.experimental import pallas as pl
from jax.experimental.pallas import tpu as pltpu
from jax.experimental.pallas import tpu_sc as plsc

_NC = 2
_NS = 16
_K = 128
_Q = 6


def _sc_aggregate(table2d, ei3, z2d, n, w):
    nch = ei3.shape[1]
    ch = nch // (_NC * _NS)
    nx = nch - ch * _NC * _NS
    chm = ch // _Q * _Q
    per = (n // _NS) // 8 * 8
    last = n - (_NS - 1) * per
    mesh = plsc.VectorSubcoreMesh(core_axis_name="c", subcore_axis_name="s")

    @functools.partial(
        pl.kernel,
        out_type=jax.ShapeDtypeStruct((_NC, n, w), jnp.float32),
        mesh=mesh,
        scratch_types=[
            pltpu.VMEM((ch + 1, _K), jnp.int32),
            pltpu.VMEM((ch + 1, _K), jnp.int32),
            pltpu.VMEM((_Q, _K, w), jnp.float32),
            pltpu.SemaphoreType.DMA((_Q,)),
            pltpu.SemaphoreType.DMA((_Q,)),
            pltpu.VMEM_SHARED((n, w), jnp.float32),
        ],
        compiler_params=pltpu.CompilerParams(use_tc_tiling_on_sc=False),
    )
    def agg(table_hbm, ei_hbm, z_hbm, out_hbm, sidx, didx,
            rows, gsem, ssem, acc):
        cid = lax.axis_index("c")
        sid = lax.axis_index("s")
        wid = cid * _NS + sid
        base = sid * per
        tbl = table_hbm
        out = out_hbm
        zr = z_hbm
        src2 = ei_hbm.at[0]
        dst2 = ei_hbm.at[1]

        @pl.when(sid < _NS - 1)
        def _():
            pltpu.sync_copy(zr.at[pl.ds(0, per)], acc.at[pl.ds(base, per)])

        @pl.when(sid == _NS - 1)
        def _():
            pltpu.sync_copy(zr, acc.at[pl.ds(base, last)])

        pltpu.sync_copy(src2.at[pl.ds(wid * ch, ch)], sidx.at[pl.ds(0, ch)])
        pltpu.sync_copy(dst2.at[pl.ds(wid * ch, ch)], didx.at[pl.ds(0, ch)])

        @pl.when(wid < nx)
        def _():
            xrow = _NC * _NS * ch + wid
            pltpu.sync_copy(src2.at[pl.ds(xrow, 1)], sidx.at[pl.ds(ch, 1)])
            pltpu.sync_copy(dst2.at[pl.ds(xrow, 1)], didx.at[pl.ds(ch, 1)])

        plsc.subcore_barrier()

        for b in range(_Q):
            pltpu.async_copy(tbl.at[sidx.at[b]], rows.at[b], gsem.at[b])

        @pl.loop(0, chm, step=_Q)
        def _(j):
            descs = []
            for b in range(_Q):
                pltpu.make_async_copy(tbl.at[sidx.at[j + b]],
                                      rows.at[b], gsem.at[b]).wait()
                descs.append(pltpu.async_copy(
                    rows.at[b], acc.at[didx.at[j + b]], ssem.at[b], add=True))
            for b in range(_Q):
                @pl.when(j + _Q + b < chm)
                def _(b=b):
                    descs[b].wait()
                    pltpu.async_copy(tbl.at[sidx.at[j + _Q + b]],
                                     rows.at[b], gsem.at[b])

        for b in range(_Q):
            pltpu.make_async_copy(rows.at[b], acc.at[didx.at[chm - _Q + b]],
                                  ssem.at[b]).wait()

        @pl.loop(chm, ch)
        def _(j):
            pltpu.sync_copy(tbl.at[sidx.at[j]], rows.at[0])
            pltpu.sync_copy(rows.at[0], acc.at[didx.at[j]], add=True)

        @pl.when(wid < nx)
        def _():
            pltpu.sync_copy(tbl.at[sidx.at[ch]], rows.at[1])
            pltpu.sync_copy(rows.at[1], acc.at[didx.at[ch]], add=True)

        plsc.subcore_barrier()

        @pl.when(sid < _NS - 1)
        def _():
            pltpu.sync_copy(acc.at[pl.ds(base, per)],
                            out.at[cid, pl.ds(base, per)])

        @pl.when(sid == _NS - 1)
        def _():
            pltpu.sync_copy(acc.at[pl.ds(base, last)],
                            out.at[cid, pl.ds(base, last)])

    return agg(table2d, ei3, z2d)


def _pack(v, f):
    n, w = v.shape
    t2 = v.reshape(n // f, f, w)
    lanes = f * w
    li = lax.broadcasted_iota(jnp.int32, (w, lanes), 1)
    ji = lax.broadcasted_iota(jnp.int32, (w, lanes), 0)
    out = None
    for q in range(f):
        sq = (li == q * w + ji).astype(jnp.float32)
        term = jnp.dot(t2[:, q, :], sq, preferred_element_type=jnp.float32)
        out = term if out is None else out + term
    return out


def _tc1_body(x_ref, wl_ref, wr_ref, table_ref, r1_ref):
    dims = (((1,), (1,)), ((), ()))
    t1 = lax.dot_general(x_ref[...], wl_ref[...], dims,
                         preferred_element_type=jnp.float32)
    r1 = lax.dot_general(x_ref[...], wr_ref[...], dims,
                         preferred_element_type=jnp.float32)
    tab = jnp.concatenate([t1, jnp.ones_like(t1)], axis=1)
    table_ref[...] = _pack(tab, 4)
    r1_ref[...] = r1


def _tc2_body(acc_ref, r1_ref, b1_ref, h_ref):
    h = r1_ref.shape[-1]
    a = acc_ref[...]
    s = a[0] + a[1]
    cnt = jnp.maximum(s[:, h:h + 1], 1.0)
    h_ref[...] = jnp.maximum(s[:, :h] / cnt + b1_ref[...][None, :]
                             + r1_ref[...], 0.0)


def _tc3_body(acc2_ref, acc1_ref, h_ref, w2l_ref, w2r_ref, b2_ref, o_ref):
    c, hd = w2l_ref.shape
    a2 = acc2_ref[...]
    s2 = a2[0] + a2[1]
    a1 = acc1_ref[...]
    cnt = jnp.maximum(a1[0, :, hd:hd + 1] + a1[1, :, hd:hd + 1], 1.0)
    hv = h_ref[...]
    dims = (((1,), (1,)), ((), ()))
    z = (lax.dot_general(s2 / cnt, w2l_ref[...], dims,
                         preferred_element_type=jnp.float32)
         + lax.dot_general(hv, w2r_ref[...], dims,
                           preferred_element_type=jnp.float32)
         + b2_ref[...][None, :])
    z = z - jnp.max(z, axis=1, keepdims=True)
    o_ref[...] = z - jnp.log(jnp.sum(jnp.exp(z), axis=1, keepdims=True))


def kernel(x, edge_index, W1_l, b1_l, W1_r, W2_l, b2_l, W2_r):
    n, d = x.shape
    h = W1_l.shape[0]
    c = W2_l.shape[0]

    e = edge_index.shape[1]
    ei3 = edge_index.reshape(2, e // _K, _K)
    zrows = n - (_NS - 1) * ((n // _NS) // 8 * 8)
    z1 = jnp.zeros((zrows, 2 * h), jnp.float32)
    z2 = jnp.zeros((zrows, h), jnp.float32)

    bn = 2000
    grid = (n // bn,)
    full = lambda shape: pl.BlockSpec(shape, lambda i: (0,) * len(shape))
    rowblk = lambda mnr: pl.BlockSpec((bn, mnr), lambda i: (i, 0))
    accblk = lambda mnr: pl.BlockSpec((_NC, bn, mnr), lambda i: (0, i, 0))

    table1, r1 = pl.pallas_call(
        _tc1_body,
        out_shape=[
            jax.ShapeDtypeStruct((n // 4, 128), jnp.float32),
            jax.ShapeDtypeStruct((n, h), jnp.float32),
        ],
    )(x, W1_l, W1_r)
    table1 = table1.reshape(n, 2 * h)

    acc1 = _sc_aggregate(table1, ei3, z1, n, 2 * h)

    hfeat = pl.pallas_call(
        _tc2_body,
        grid=grid,
        in_specs=[accblk(2 * h), rowblk(h), full((h,))],
        out_specs=rowblk(h),
        out_shape=jax.ShapeDtypeStruct((n, h), jnp.float32),
    )(acc1, r1, b1_l)

    acc2 = _sc_aggregate(hfeat, ei3, z2, n, h)

    out = pl.pallas_call(
        _tc3_body,
        grid=grid,
        in_specs=[accblk(h), accblk(2 * h), rowblk(h),
                  full((c, h)), full((c, h)), full((c,))],
        out_specs=rowblk(c),
        out_shape=jax.ShapeDtypeStruct((n, c), jnp.float32),
    )(acc2, acc1, hfeat, W2_l, W2_r, b2_l)

    return out
